# Initial kernel scaffold; baseline (speedup 1.0000x reference)
import jax
import jax.numpy as jnp
from jax.experimental import pallas as pl


def kernel(topic_reps, word_reps, topic_lens, para_spans, x_spans, shell_spans):
    raise NotImplementedError("write your pallas kernel here")


# speedup vs baseline: 191.3421x; 191.3421x over previous
import functools

import jax
import jax.numpy as jnp
from jax import lax
from jax.experimental import pallas as pl
from jax.experimental.pallas import tpu as pltpu
from jax.experimental.pallas import tpu_sc as plsc

B = 16
T_SEQ = 512
W_SEQ = 2048
H = 512
NS = 64

NC = 2
NSUB = 16
L = 16
NW = NC * NSUB
SPANS = B * NS
SPW = SPANS // NW
VPR = H // L


def _sc_body(word_hbm, topic_hbm, lens_hbm, s0_hbm, s1_hbm, s2_hbm,
             out0_hbm, out1_hbm, out2_hbm, tout_hbm,
             spans_v, idx_a, idx_b, idx_c, idx_d,
             buf_a, buf_b, buf_c, buf_d,
             lens_v, idx_t, buf_t,
             sem_a, sem_b, sem_c, sem_d, sem_t):
    cid = lax.axis_index("c")
    sid = lax.axis_index("s")
    wid = sid * NC + cid
    lane = lax.iota(jnp.int32, L)

    def do_set(spans_hbm, out_hbm):
        base_el = pl.multiple_of(wid * (SPW * 3), 8)
        pltpu.sync_copy(spans_hbm.at[pl.ds(base_el, SPW * 3)], spans_v)
        for j in range(SPW // L):
            off = (j * L + lane) * 3
            e = plsc.load_gather(spans_v, [off])
            s = plsc.load_gather(spans_v, [off + 1])
            t = plsc.load_gather(spans_v, [off + 2])
            sm1 = s - 1
            sm1 = jnp.where(sm1 < 0, sm1 + W_SEQ, sm1)
            t1 = jnp.minimum(t + 1, W_SEQ - 1)
            rowbase = e * W_SEQ
            sl = pl.ds(j * L, L)
            idx_a[sl] = (rowbase + sm1) * 2
            idx_b[sl] = (rowbase + t) * 2
            idx_c[sl] = (rowbase + t1) * 2 + 1
            idx_d[sl] = (rowbase + s) * 2 + 1
        ca = pltpu.async_copy(word_hbm.at[idx_a], buf_a, sem_a)
        cb = pltpu.async_copy(word_hbm.at[idx_b], buf_b, sem_b)
        cc = pltpu.async_copy(word_hbm.at[idx_c], buf_c, sem_c)
        cd = pltpu.async_copy(word_hbm.at[idx_d], buf_d, sem_d)
        ca.wait()
        cb.wait()
        cc.wait()
        cd.wait()

        def diff_body(i, carry):
            for o in range(VPR):
                hsl = pl.ds(o * L, L)
                buf_b[i, hsl] = buf_b[i, hsl] - buf_a[i, hsl]
                buf_d[i, hsl] = buf_d[i, hsl] - buf_c[i, hsl]
            return carry

        lax.fori_loop(0, SPW, diff_body, 0)

        rbase = pl.multiple_of(wid * SPW, 8)
        pltpu.sync_copy(buf_b, out_hbm.at[pl.ds(rbase, SPW), 0])
        pltpu.sync_copy(buf_d, out_hbm.at[pl.ds(rbase, SPW), 1])
        pltpu.sync_copy(buf_a, out_hbm.at[pl.ds(rbase, SPW), 2])
        pltpu.sync_copy(buf_c, out_hbm.at[pl.ds(rbase, SPW), 3])

    do_set(s0_hbm, out0_hbm)
    do_set(s1_hbm, out1_hbm)
    do_set(s2_hbm, out2_hbm)

    @pl.when(wid == 0)
    def _topic():
        pltpu.sync_copy(lens_hbm, lens_v)
        for j in range(2 * B // L):
            gl = j * L + lane
            i = gl // 2
            par = gl % 2
            ln = plsc.load_gather(lens_v, [i])
            lm1 = ln - 1
            lm1 = jnp.where(lm1 < 0, lm1 + T_SEQ, lm1)
            r_fwd = (i * T_SEQ + lm1) * 2
            r_bwd = i * T_SEQ * 2 + 1
            idx_t[pl.ds(j * L, L)] = jnp.where(par == 0, r_fwd, r_bwd)
        pltpu.async_copy(topic_hbm.at[idx_t], buf_t, sem_t).wait()
        pltpu.sync_copy(buf_t, tout_hbm)


_mesh = plsc.VectorSubcoreMesh(core_axis_name="c", subcore_axis_name="s")

_sc_call = functools.partial(
    pl.kernel,
    mesh=_mesh,
    compiler_params=pltpu.CompilerParams(
        needs_layout_passes=False,
        use_tc_tiling_on_sc=False,
    ),
    out_type=(
        jax.ShapeDtypeStruct((SPANS, 4, H), jnp.float32),
        jax.ShapeDtypeStruct((SPANS, 4, H), jnp.float32),
        jax.ShapeDtypeStruct((SPANS, 4, H), jnp.float32),
        jax.ShapeDtypeStruct((2 * B, H), jnp.float32),
    ),
    scratch_types=[
        pltpu.VMEM((SPW * 3,), jnp.int32),
        pltpu.VMEM((SPW,), jnp.int32),
        pltpu.VMEM((SPW,), jnp.int32),
        pltpu.VMEM((SPW,), jnp.int32),
        pltpu.VMEM((SPW,), jnp.int32),
        pltpu.VMEM((SPW, H), jnp.float32),
        pltpu.VMEM((SPW, H), jnp.float32),
        pltpu.VMEM((SPW, H), jnp.float32),
        pltpu.VMEM((SPW, H), jnp.float32),
        pltpu.VMEM((B,), jnp.int32),
        pltpu.VMEM((2 * B,), jnp.int32),
        pltpu.VMEM((2 * B, H), jnp.float32),
        pltpu.SemaphoreType.DMA,
        pltpu.SemaphoreType.DMA,
        pltpu.SemaphoreType.DMA,
        pltpu.SemaphoreType.DMA,
        pltpu.SemaphoreType.DMA,
    ],
)(_sc_body)


@jax.jit
def kernel(topic_reps, word_reps, topic_lens, para_spans, x_spans, shell_spans):
    word_view = word_reps.reshape(B * W_SEQ * 2, H)
    topic_view = topic_reps.reshape(B * T_SEQ * 2, H)
    lens = topic_lens.astype(jnp.int32)
    s0 = para_spans.astype(jnp.int32).reshape(-1)
    s1 = x_spans.astype(jnp.int32).reshape(-1)
    s2 = shell_spans.astype(jnp.int32).reshape(-1)
    o_para, o_adu, o_shell, o_topic = _sc_call(
        word_view, topic_view, lens, s0, s1, s2)
    para_reps = o_para.reshape(B, NS, 4 * H)
    adu_reps = o_adu.reshape(B, NS, 4 * H)
    span_reps = o_shell.reshape(B, NS, 4 * H)
    topic_out = o_topic.reshape(B, 2 * H)
    return (topic_out, para_reps, span_reps, adu_reps)
